# baseline (speedup 1.0000x reference)
import functools

import jax
import jax.numpy as jnp
from jax import lax
from jax.experimental import pallas as pl
from jax.experimental.pallas import tpu as pltpu
from jax.experimental.pallas import tpu_sc as plsc

N = 512
H = 256
A = 16
SCALE = float(N) / (A ** 0.5)

ROWS_SC = 256
NW = 32
RPW = ROWS_SC // NW
CH = 128
NCH = N // CH
VL = 16
NV = H // VL

BI = 8


def _proj_body(t_ref, w1_ref, w2_ref, b2_ref, out_ref):
    tp = lax.dot_general(t_ref[...], w2_ref[...], (((1,), (1,)), ((), ())),
                         preferred_element_type=jnp.float32) + b2_ref[...]
    u = lax.dot_general(tp, w1_ref[...], (((1,), (0,)), ((), ())),
                        preferred_element_type=jnp.float32)
    out_ref[...] = u * jnp.float32(SCALE)


def _sc_attn(sp_hbm, u_hbm, out_hbm, u_v, buf_a, buf_b, out_v, acc_v,
             sem_a, sem_b):
    cid = lax.axis_index("c")
    sid = lax.axis_index("s")
    wid = sid * 2 + cid
    base = wid * RPW
    pltpu.sync_copy(u_hbm.at[pl.ds(base, RPW)], u_v)
    dc = base // CH

    bufs = [(buf_a, sem_a), (buf_b, sem_b)]

    def dma(r, c):
        buf, sem = bufs[c % 2]
        return pltpu.make_async_copy(
            sp_hbm.at[base + r, pl.ds(c * CH, CH), :], buf, sem)

    dma(0, 0).start()

    zeros = jnp.zeros((VL,), jnp.float32)

    lane = lax.iota(jnp.int32, VL)
    perms = [(lane ^ (1 << b)).reshape(VL, 1) for b in range(4)]
    dnums = lax.GatherDimensionNumbers(
        offset_dims=(), collapsed_slice_dims=(0,), start_index_map=(0,))

    def hsum(v):
        for p in perms:
            v = v + lax.gather(v, p, dnums, slice_sizes=(1,),
                               mode=lax.GatherScatterMode.PROMISE_IN_BOUNDS)
        return v

    def row_body(r, _):
        i_glob = base + r
        uv = [u_v[r, pl.ds(VL * k, VL)] for k in range(NV)]
        for b in range(2):
            for k in range(NV + 1):
                acc_v[b, k] = zeros
        for c in range(NCH):
            buf, sem = bufs[c % 2]
            if c + 1 < NCH:
                dma(r, c + 1).start()
            else:
                @pl.when(r + 1 < RPW)
                def _():
                    dma(r + 1, 0).start()
            dma(r, c).wait()

            @plsc.parallel_loop(0, CH, step=2, unroll=2)
            def _(j0):
                for b in range(2):
                    j = j0 + b
                    x = [buf[j, pl.ds(VL * k, VL)] for k in range(NV)]
                    s = [x[k] * uv[k] for k in range(8)]
                    for k in range(8, NV):
                        s[k - 8] = s[k - 8] + x[k] * uv[k]
                    s4 = [(s[2 * k] + s[2 * k + 1]) for k in range(4)]
                    ev = jnp.exp(hsum((s4[0] + s4[1]) + (s4[2] + s4[3])))
                    for k in range(12, NV):
                        plsc.addupdate(acc_v.at[b, k], ev * x[k])
                    plsc.addupdate(acc_v.at[b, NV], ev)
                    for k in range(12):
                        plsc.addupdate(acc_v.at[b, k],
                                       ev * buf[j, pl.ds(VL * k, VL)])

            flag = (dc == c).astype(jnp.float32)
            jd = i_glob - c * CH
            jd = lax.max(0, lax.min(jd, CH - 1))
            xd = [buf[jd, pl.ds(VL * k, VL)] for k in range(NV)]
            sd = xd[0] * uv[0]
            for k in range(1, NV):
                sd = sd + xd[k] * uv[k]
            fv = jnp.full((VL,), flag)
            evd = jnp.exp(hsum(sd) * fv) * fv
            for k in range(NV):
                plsc.addupdate(acc_v.at[0, k], -(evd * xd[k]))
            plsc.addupdate(acc_v.at[0, NV], -evd)

        den = acc_v[0, NV] + acc_v[1, NV]
        for k in range(NV):
            out_v[r, pl.ds(VL * k, VL)] = (acc_v[0, k] + acc_v[1, k]) / den
        return 0

    lax.fori_loop(0, RPW, row_body, 0)
    pltpu.sync_copy(out_v, out_hbm.at[pl.ds(base, RPW)])


def _tc_attn_body(sp_ref, tp_ref, w1_ref, b1_ref, w2_ref, b2_ref, out_ref):
    tp = lax.dot_general(tp_ref[...], w2_ref[...],
                         (((1,), (1,)), ((), ())),
                         preferred_element_type=jnp.float32) + b2_ref[...]
    u = lax.dot_general(tp, w1_ref[...], (((1,), (0,)), ((), ())),
                        preferred_element_type=jnp.float32)
    c = lax.dot_general(b1_ref[...], tp, (((1,), (1,)), ((), ())),
                        preferred_element_type=jnp.float32)
    i0 = ROWS_SC + pl.program_id(0) * BI

    spf = sp_ref[...].reshape(BI * N, H)
    score = lax.dot_general(spf, u, (((1,), (1,)), ((), ())),
                            preferred_element_type=jnp.float32)
    score = (score + c) * jnp.float32(SCALE)
    e = jnp.exp(score)
    row = lax.broadcasted_iota(jnp.int32, (BI * N, BI), 0)
    col = lax.broadcasted_iota(jnp.int32, (BI * N, BI), 1)
    keep = (row // N == col) & (row % N != i0 + col)
    e = jnp.where(keep, e, 0.0)
    num = lax.dot_general(e, spf, (((0,), (0,)), ((), ())),
                          preferred_element_type=jnp.float32)
    ones = jnp.ones((BI * N, 1), dtype=jnp.float32)
    den = lax.dot_general(e, ones, (((0,), (0,)), ((), ())),
                          preferred_element_type=jnp.float32)
    out_ref[...] = num / den


def kernel(spatial_ht, temporal_ht, ts_mask, same_scene_mask, W1, b1, W2, b2):
    del ts_mask, same_scene_mask
    b1r = b1.reshape(1, A)
    b2r = b2.reshape(1, A)

    u = pl.pallas_call(
        _proj_body,
        out_shape=jax.ShapeDtypeStruct((ROWS_SC, H), jnp.float32),
    )(temporal_ht[:ROWS_SC], W1, W2, b2r)

    mesh = plsc.VectorSubcoreMesh(core_axis_name="c", subcore_axis_name="s")
    sc_attn = pl.kernel(
        _sc_attn,
        out_type=jax.ShapeDtypeStruct((ROWS_SC, H), jnp.float32),
        mesh=mesh,
        scratch_types=[
            pltpu.VMEM((RPW, H), jnp.float32),
            pltpu.VMEM((CH, H), jnp.float32),
            pltpu.VMEM((CH, H), jnp.float32),
            pltpu.VMEM((RPW, H), jnp.float32),
            pltpu.VMEM((2, NV + 1, VL), jnp.float32),
            pltpu.SemaphoreType.DMA,
            pltpu.SemaphoreType.DMA,
        ],
    )
    out_sc = sc_attn(spatial_ht, u)

    grid = (N - ROWS_SC) // BI
    off = ROWS_SC // BI
    out_tc = pl.pallas_call(
        _tc_attn_body,
        grid=(grid,),
        in_specs=[
            pl.BlockSpec((BI, N, H), lambda i: (i + off, 0, 0)),
            pl.BlockSpec((BI, H), lambda i: (i + off, 0)),
            pl.BlockSpec((A, H), lambda i: (0, 0)),
            pl.BlockSpec((1, A), lambda i: (0, 0)),
            pl.BlockSpec((A, H), lambda i: (0, 0)),
            pl.BlockSpec((1, A), lambda i: (0, 0)),
        ],
        out_specs=pl.BlockSpec((BI, H), lambda i: (i, 0)),
        out_shape=jax.ShapeDtypeStruct((N - ROWS_SC, H), jnp.float32),
    )(spatial_ht, temporal_ht, W1, b1r, W2, b2r)

    return jnp.concatenate([out_sc, out_tc], axis=0)


# speedup vs baseline: 1.4079x; 1.4079x over previous
import functools

import jax
import jax.numpy as jnp
from jax import lax
from jax.experimental import pallas as pl
from jax.experimental.pallas import tpu as pltpu
from jax.experimental.pallas import tpu_sc as plsc

N = 512
H = 256
A = 16
SCALE = float(N) / (A ** 0.5)

ROWS_SC = 192
NW = 32
RPW = ROWS_SC // NW
CH = 128
NCH = N // CH
VL = 16
NV = H // VL

BI = 8


def _proj_body(t_ref, w1_ref, w2_ref, b2_ref, out_ref):
    tp = lax.dot_general(t_ref[...], w2_ref[...], (((1,), (1,)), ((), ())),
                         preferred_element_type=jnp.float32) + b2_ref[...]
    u = lax.dot_general(tp, w1_ref[...], (((1,), (0,)), ((), ())),
                        preferred_element_type=jnp.float32)
    out_ref[...] = u * jnp.float32(SCALE)


def _sc_attn(sp_hbm, u_hbm, out_hbm, u_v, buf_a, buf_b, out_v, acc_v,
             sem_a, sem_b):
    cid = lax.axis_index("c")
    sid = lax.axis_index("s")
    wid = sid * 2 + cid
    base = wid * RPW
    pltpu.sync_copy(u_hbm.at[wid], u_v)

    bufs = [(buf_a, sem_a), (buf_b, sem_b)]

    def dma(r, c):
        buf, sem = bufs[c % 2]
        return pltpu.make_async_copy(
            sp_hbm.at[base + r, pl.ds(c * CH, CH), :], buf, sem)

    dma(0, 0).start()

    zeros = jnp.zeros((VL,), jnp.float32)

    lane = lax.iota(jnp.int32, VL)
    perms = [(lane ^ (1 << b)).reshape(VL, 1) for b in range(4)]
    dnums = lax.GatherDimensionNumbers(
        offset_dims=(), collapsed_slice_dims=(0,), start_index_map=(0,))

    def hsum(v):
        for p in perms:
            v = v + lax.gather(v, p, dnums, slice_sizes=(1,),
                               mode=lax.GatherScatterMode.PROMISE_IN_BOUNDS)
        return v

    def row_body(r, _):
        i_glob = base + r
        uv = [u_v[r, pl.ds(VL * k, VL)] for k in range(NV)]
        for b in range(2):
            for k in range(NV + 1):
                acc_v[b, k] = zeros
        for c in range(NCH):
            buf, sem = bufs[c % 2]
            if c + 1 < NCH:
                dma(r, c + 1).start()
            else:
                @pl.when(r + 1 < RPW)
                def _():
                    dma(r + 1, 0).start()
            dma(r, c).wait()

            @plsc.parallel_loop(0, CH, step=1, unroll=3)
            def _(j):
                x = [buf[j, pl.ds(VL * k, VL)] for k in range(NV)]
                s = [x[k] * uv[k] for k in range(8)]
                for k in range(8, NV):
                    s[k - 8] = s[k - 8] + x[k] * uv[k]
                s4 = [(s[2 * k] + s[2 * k + 1]) for k in range(4)]
                ev = jnp.exp(hsum((s4[0] + s4[1]) + (s4[2] + s4[3])))
                for k in range(12, NV):
                    plsc.addupdate(acc_v.at[0, k], ev * x[k])
                plsc.addupdate(acc_v.at[0, NV], ev)
                for k in range(12):
                    plsc.addupdate(acc_v.at[0, k],
                                   ev * buf[j, pl.ds(VL * k, VL)])

            flag = (i_glob // CH == c).astype(jnp.float32)
            jd = i_glob - c * CH
            jd = lax.max(0, lax.min(jd, CH - 1))
            xd = [buf[jd, pl.ds(VL * k, VL)] for k in range(NV)]
            sd = xd[0] * uv[0]
            for k in range(1, NV):
                sd = sd + xd[k] * uv[k]
            fv = jnp.full((VL,), flag)
            evd = jnp.exp(hsum(sd) * fv) * fv
            for k in range(NV):
                plsc.addupdate(acc_v.at[0, k], -(evd * xd[k]))
            plsc.addupdate(acc_v.at[0, NV], -evd)

        den = acc_v[0, NV] + acc_v[1, NV]
        for k in range(NV):
            out_v[r, pl.ds(VL * k, VL)] = (acc_v[0, k] + acc_v[1, k]) / den
        return 0

    lax.fori_loop(0, RPW, row_body, 0)
    pltpu.sync_copy(out_v, out_hbm.at[wid])


def _tc_attn_body(sp_ref, tp_ref, w1_ref, b1_ref, w2_ref, b2_ref, out_ref):
    tp = lax.dot_general(tp_ref[...], w2_ref[...],
                         (((1,), (1,)), ((), ())),
                         preferred_element_type=jnp.float32) + b2_ref[...]
    u = lax.dot_general(tp, w1_ref[...], (((1,), (0,)), ((), ())),
                        preferred_element_type=jnp.float32)
    c = lax.dot_general(b1_ref[...], tp, (((1,), (1,)), ((), ())),
                        preferred_element_type=jnp.float32)
    i0 = ROWS_SC + pl.program_id(0) * BI

    spf = sp_ref[...].reshape(BI * N, H)
    score = lax.dot_general(spf, u, (((1,), (1,)), ((), ())),
                            preferred_element_type=jnp.float32)
    score = (score + c) * jnp.float32(SCALE)
    e = jnp.exp(score)
    row = lax.broadcasted_iota(jnp.int32, (BI * N, BI), 0)
    col = lax.broadcasted_iota(jnp.int32, (BI * N, BI), 1)
    keep = (row // N == col) & (row % N != i0 + col)
    e = jnp.where(keep, e, 0.0)
    num = lax.dot_general(e, spf, (((0,), (0,)), ((), ())),
                          preferred_element_type=jnp.float32)
    ones = jnp.ones((BI * N, 1), dtype=jnp.float32)
    den = lax.dot_general(e, ones, (((0,), (0,)), ((), ())),
                          preferred_element_type=jnp.float32)
    out_ref[...] = num / den


def kernel(spatial_ht, temporal_ht, ts_mask, same_scene_mask, W1, b1, W2, b2):
    del ts_mask, same_scene_mask
    b1r = b1.reshape(1, A)
    b2r = b2.reshape(1, A)

    u = pl.pallas_call(
        _proj_body,
        out_shape=jax.ShapeDtypeStruct((ROWS_SC, H), jnp.float32),
    )(temporal_ht[:ROWS_SC], W1, W2, b2r)

    mesh = plsc.VectorSubcoreMesh(core_axis_name="c", subcore_axis_name="s")
    sc_attn = pl.kernel(
        _sc_attn,
        out_type=jax.ShapeDtypeStruct((NW, RPW, H), jnp.float32),
        mesh=mesh,
        scratch_types=[
            pltpu.VMEM((RPW, H), jnp.float32),
            pltpu.VMEM((CH, H), jnp.float32),
            pltpu.VMEM((CH, H), jnp.float32),
            pltpu.VMEM((RPW, H), jnp.float32),
            pltpu.VMEM((2, NV + 1, VL), jnp.float32),
            pltpu.SemaphoreType.DMA,
            pltpu.SemaphoreType.DMA,
        ],
    )
    out_sc = sc_attn(spatial_ht, u.reshape(NW, RPW, H)).reshape(ROWS_SC, H)

    grid = (N - ROWS_SC) // BI
    off = ROWS_SC // BI
    out_tc = pl.pallas_call(
        _tc_attn_body,
        grid=(grid,),
        in_specs=[
            pl.BlockSpec((BI, N, H), lambda i: (i + off, 0, 0)),
            pl.BlockSpec((BI, H), lambda i: (i + off, 0)),
            pl.BlockSpec((A, H), lambda i: (0, 0)),
            pl.BlockSpec((1, A), lambda i: (0, 0)),
            pl.BlockSpec((A, H), lambda i: (0, 0)),
            pl.BlockSpec((1, A), lambda i: (0, 0)),
        ],
        out_specs=pl.BlockSpec((BI, H), lambda i: (i, 0)),
        out_shape=jax.ShapeDtypeStruct((N - ROWS_SC, H), jnp.float32),
    )(spatial_ht, temporal_ht, W1, b1r, W2, b2r)

    return jnp.concatenate([out_sc, out_tc], axis=0)


# speedup vs baseline: 1.4437x; 1.0254x over previous
import functools

import jax
import jax.numpy as jnp
from jax import lax
from jax.experimental import pallas as pl
from jax.experimental.pallas import tpu as pltpu
from jax.experimental.pallas import tpu_sc as plsc

N = 512
H = 256
A = 16
SCALE = float(N) / (A ** 0.5)

ROWS_SC = 160
NW = 32
RPW = ROWS_SC // NW
CH = 128
NCH = N // CH
VL = 16
NV = H // VL

BI = 8


def _proj_body(t_ref, w1_ref, w2_ref, b2_ref, out_ref):
    tp = lax.dot_general(t_ref[...], w2_ref[...], (((1,), (1,)), ((), ())),
                         preferred_element_type=jnp.float32) + b2_ref[...]
    u = lax.dot_general(tp, w1_ref[...], (((1,), (0,)), ((), ())),
                        preferred_element_type=jnp.float32)
    out_ref[...] = u * jnp.float32(SCALE)


def _sc_attn(sp_hbm, u_hbm, out_hbm, u_v, buf_a, buf_b, out_v, acc_v,
             sem_a, sem_b):
    cid = lax.axis_index("c")
    sid = lax.axis_index("s")
    wid = sid * 2 + cid
    base = wid * RPW
    pltpu.sync_copy(u_hbm.at[wid], u_v)

    bufs = [(buf_a, sem_a), (buf_b, sem_b)]

    def dma(r, c):
        buf, sem = bufs[c % 2]
        return pltpu.make_async_copy(
            sp_hbm.at[base + r, pl.ds(c * CH, CH), :], buf, sem)

    dma(0, 0).start()

    zeros = jnp.zeros((VL,), jnp.float32)

    lane = lax.iota(jnp.int32, VL)
    perms = [(lane ^ (1 << b)).reshape(VL, 1) for b in range(4)]
    dnums = lax.GatherDimensionNumbers(
        offset_dims=(), collapsed_slice_dims=(0,), start_index_map=(0,))

    def hsum(v):
        for p in perms:
            v = v + lax.gather(v, p, dnums, slice_sizes=(1,),
                               mode=lax.GatherScatterMode.PROMISE_IN_BOUNDS)
        return v

    def row_body(r, _):
        i_glob = base + r
        uv = [u_v[r, pl.ds(VL * k, VL)] for k in range(NV)]
        for b in range(2):
            for k in range(NV + 1):
                acc_v[b, k] = zeros
        for c in range(NCH):
            buf, sem = bufs[c % 2]
            if c + 1 < NCH:
                dma(r, c + 1).start()
            else:
                @pl.when(r + 1 < RPW)
                def _():
                    dma(r + 1, 0).start()
            dma(r, c).wait()

            @plsc.parallel_loop(0, CH, step=1, unroll=3)
            def _(j):
                x = [buf[j, pl.ds(VL * k, VL)] for k in range(NV)]
                s = [x[k] * uv[k] for k in range(8)]
                for k in range(8, NV):
                    s[k - 8] = s[k - 8] + x[k] * uv[k]
                s4 = [(s[2 * k] + s[2 * k + 1]) for k in range(4)]
                ev = jnp.exp(hsum((s4[0] + s4[1]) + (s4[2] + s4[3])))
                for k in range(12, NV):
                    plsc.addupdate(acc_v.at[0, k], ev * x[k])
                plsc.addupdate(acc_v.at[0, NV], ev)
                for k in range(12):
                    plsc.addupdate(acc_v.at[0, k],
                                   ev * buf[j, pl.ds(VL * k, VL)])

            flag = (i_glob // CH == c).astype(jnp.float32)
            jd = i_glob - c * CH
            jd = lax.max(0, lax.min(jd, CH - 1))
            xd = [buf[jd, pl.ds(VL * k, VL)] for k in range(NV)]
            sd = xd[0] * uv[0]
            for k in range(1, NV):
                sd = sd + xd[k] * uv[k]
            fv = jnp.full((VL,), flag)
            evd = jnp.exp(hsum(sd) * fv) * fv
            for k in range(NV):
                plsc.addupdate(acc_v.at[0, k], -(evd * xd[k]))
            plsc.addupdate(acc_v.at[0, NV], -evd)

        den = acc_v[0, NV] + acc_v[1, NV]
        for k in range(NV):
            out_v[r, pl.ds(VL * k, VL)] = (acc_v[0, k] + acc_v[1, k]) / den
        return 0

    lax.fori_loop(0, RPW, row_body, 0)
    pltpu.sync_copy(out_v, out_hbm.at[wid])


def _tc_attn_body(sp_ref, tp_ref, w1_ref, b1_ref, w2_ref, b2_ref, out_ref):
    tp = lax.dot_general(tp_ref[...], w2_ref[...],
                         (((1,), (1,)), ((), ())),
                         preferred_element_type=jnp.float32) + b2_ref[...]
    u = lax.dot_general(tp, w1_ref[...], (((1,), (0,)), ((), ())),
                        preferred_element_type=jnp.float32)
    c = lax.dot_general(b1_ref[...], tp, (((1,), (1,)), ((), ())),
                        preferred_element_type=jnp.float32)
    i0 = ROWS_SC + pl.program_id(0) * BI

    spf = sp_ref[...].reshape(BI * N, H)
    score = lax.dot_general(spf, u, (((1,), (1,)), ((), ())),
                            preferred_element_type=jnp.float32)
    score = (score + c) * jnp.float32(SCALE)
    e = jnp.exp(score)
    row = lax.broadcasted_iota(jnp.int32, (BI * N, BI), 0)
    col = lax.broadcasted_iota(jnp.int32, (BI * N, BI), 1)
    keep = (row // N == col) & (row % N != i0 + col)
    e = jnp.where(keep, e, 0.0)
    num = lax.dot_general(e, spf, (((0,), (0,)), ((), ())),
                          preferred_element_type=jnp.float32)
    ones = jnp.ones((BI * N, 1), dtype=jnp.float32)
    den = lax.dot_general(e, ones, (((0,), (0,)), ((), ())),
                          preferred_element_type=jnp.float32)
    out_ref[...] = num / den


def kernel(spatial_ht, temporal_ht, ts_mask, same_scene_mask, W1, b1, W2, b2):
    del ts_mask, same_scene_mask
    b1r = b1.reshape(1, A)
    b2r = b2.reshape(1, A)

    u = pl.pallas_call(
        _proj_body,
        out_shape=jax.ShapeDtypeStruct((ROWS_SC, H), jnp.float32),
    )(temporal_ht[:ROWS_SC], W1, W2, b2r)

    mesh = plsc.VectorSubcoreMesh(core_axis_name="c", subcore_axis_name="s")
    sc_attn = pl.kernel(
        _sc_attn,
        out_type=jax.ShapeDtypeStruct((NW, RPW, H), jnp.float32),
        mesh=mesh,
        scratch_types=[
            pltpu.VMEM((RPW, H), jnp.float32),
            pltpu.VMEM((CH, H), jnp.float32),
            pltpu.VMEM((CH, H), jnp.float32),
            pltpu.VMEM((RPW, H), jnp.float32),
            pltpu.VMEM((2, NV + 1, VL), jnp.float32),
            pltpu.SemaphoreType.DMA,
            pltpu.SemaphoreType.DMA,
        ],
    )
    out_sc = sc_attn(spatial_ht, u.reshape(NW, RPW, H)).reshape(ROWS_SC, H)

    grid = (N - ROWS_SC) // BI
    off = ROWS_SC // BI
    out_tc = pl.pallas_call(
        _tc_attn_body,
        grid=(grid,),
        in_specs=[
            pl.BlockSpec((BI, N, H), lambda i: (i + off, 0, 0)),
            pl.BlockSpec((BI, H), lambda i: (i + off, 0)),
            pl.BlockSpec((A, H), lambda i: (0, 0)),
            pl.BlockSpec((1, A), lambda i: (0, 0)),
            pl.BlockSpec((A, H), lambda i: (0, 0)),
            pl.BlockSpec((1, A), lambda i: (0, 0)),
        ],
        out_specs=pl.BlockSpec((BI, H), lambda i: (i, 0)),
        out_shape=jax.ShapeDtypeStruct((N - ROWS_SC, H), jnp.float32),
    )(spatial_ht, temporal_ht, W1, b1r, W2, b2r)

    return jnp.concatenate([out_sc, out_tc], axis=0)
